# Initial kernel scaffold; baseline (speedup 1.0000x reference)
import functools

import jax
import jax.numpy as jnp
from jax import lax
from jax.experimental import pallas as pl
from jax.experimental.pallas import tpu as pltpu
from jax.experimental.pallas import tpu_sc as plsc

_N = 10000
_E = 160000
_D = 256
_H = 128
_ED = 16

_EB = 8000
_NB = 1000

_NSUB = 16
_B = 80
_EPW = _E // _NSUB
_NCH = _EPW // _B
_RPT = _N // _NSUB
_ZR = 125


def _edge_encoder_body(ea_ref, we_ref, be_ref, out_ref):
    ee = jnp.dot(ea_ref[...], we_ref[...], preferred_element_type=jnp.float32)
    out_ref[0] = ee + be_ref[...]


def _edge_encoder(edge_attr, W_e, b_e):
    return pl.pallas_call(
        _edge_encoder_body,
        grid=(2, _E // _EB),
        in_specs=[
            pl.BlockSpec((_EB, _ED), lambda h, i: (i, 0)),
            pl.BlockSpec((_ED, _H), lambda h, i: (0, h)),
            pl.BlockSpec((1, _H), lambda h, i: (0, h)),
        ],
        out_specs=pl.BlockSpec((1, _EB, _H), lambda h, i: (h, i, 0)),
        out_shape=jax.ShapeDtypeStruct((2, _E, _H), jnp.float32),
    )(edge_attr, W_e, b_e.reshape(1, _D))


_sc_mesh = plsc.VectorSubcoreMesh(core_axis_name="c", subcore_axis_name="s")


@functools.partial(
    pl.kernel,
    mesh=_sc_mesh,
    out_type=jax.ShapeDtypeStruct((2 * _N, _H), jnp.float32),
    scratch_types=[
        pltpu.VMEM((_B,), jnp.int32),
        pltpu.VMEM((_B,), jnp.int32),
        pltpu.VMEM((_B, _H), jnp.float32),
        pltpu.VMEM((_B, _H), jnp.float32),
        pltpu.VMEM((_ZR, _H), jnp.float32),
        pltpu.VMEM_SHARED((_N, _H), jnp.float32),
        pltpu.SemaphoreType.DMA,
    ],
)
def _sc_aggregate(xcat, ecat, src, dst, out, srcv, dstv, gbuf, ebuf, zbuf, acc, sem):
    c = lax.axis_index("c")
    s = lax.axis_index("s")

    def _zrow(r, _):
        for j in range(_H // 16):
            zbuf[r, pl.ds(j * 16, 16)] = jnp.zeros((16,), jnp.float32)
        return 0

    lax.fori_loop(0, _ZR, _zrow, 0)
    for k in range(_RPT // _ZR):
        pltpu.sync_copy(zbuf, acc.at[pl.ds(s * _RPT + k * _ZR, _ZR)])
    plsc.subcore_barrier()

    base_e = s * _EPW
    row_off = c * _N

    def _chunk(i, _):
        eb = base_e + i * _B
        pltpu.sync_copy(src.at[pl.ds(eb, _B)], srcv)
        pltpu.sync_copy(dst.at[pl.ds(eb, _B)], dstv)
        for j in range(_B // 16):
            sl = pl.ds(j * 16, 16)
            srcv[sl] = srcv[sl] + row_off
        gath = pltpu.async_copy(xcat.at[srcv], gbuf, sem)
        pltpu.sync_copy(ecat.at[pl.ds(c * _E + eb, _B)], ebuf)
        gath.wait()

        def _row(r, _):
            for j in range(_H // 16):
                sl = pl.ds(j * 16, 16)
                gbuf[r, sl] = jnp.maximum(gbuf[r, sl] + ebuf[r, sl], 0.0)
            return 0

        lax.fori_loop(0, _B, _row, 0)
        pltpu.sync_copy(gbuf, acc.at[dstv], add=True)
        return 0

    lax.fori_loop(0, _NCH, _chunk, 0)
    plsc.subcore_barrier()
    pltpu.sync_copy(
        acc.at[pl.ds(s * _RPT, _RPT)], out.at[pl.ds(row_off + s * _RPT, _RPT)]
    )


def _mlp1_body(x_ref, a0_ref, a1_ref, eps_ref, w1_ref, b1_ref, h1_ref, sums_ref, acc_ref):
    i = pl.program_id(0)
    eps = eps_ref[0, 0]
    agg = jnp.concatenate([a0_ref[...], a1_ref[...]], axis=1)
    h = (1.0 + eps) * x_ref[...] + agg
    h1 = jnp.dot(h, w1_ref[...], preferred_element_type=jnp.float32) + b1_ref[...]
    h1_ref[...] = h1

    @pl.when(i == 0)
    def _():
        acc_ref[...] = jnp.zeros_like(acc_ref)

    acc_ref[0:1, :] += jnp.sum(h1, axis=0, keepdims=True)
    acc_ref[1:2, :] += jnp.sum(h1 * h1, axis=0, keepdims=True)

    @pl.when(i == pl.num_programs(0) - 1)
    def _():
        sums_ref[...] = acc_ref[...]


def _mlp_stage1(x, aggcat, eps_p, W1, b1):
    nb = _N // _NB
    return pl.pallas_call(
        _mlp1_body,
        grid=(nb,),
        in_specs=[
            pl.BlockSpec((_NB, _D), lambda i: (i, 0)),
            pl.BlockSpec((_NB, _H), lambda i: (i, 0)),
            pl.BlockSpec((_NB, _H), lambda i: (i + nb, 0)),
            pl.BlockSpec(memory_space=pltpu.SMEM),
            pl.BlockSpec((_D, 2 * _D), lambda i: (0, 0)),
            pl.BlockSpec((1, 2 * _D), lambda i: (0, 0)),
        ],
        out_specs=[
            pl.BlockSpec((_NB, 2 * _D), lambda i: (i, 0)),
            pl.BlockSpec((2, 2 * _D), lambda i: (0, 0)),
        ],
        out_shape=[
            jax.ShapeDtypeStruct((_N, 2 * _D), jnp.float32),
            jax.ShapeDtypeStruct((2, 2 * _D), jnp.float32),
        ],
        scratch_shapes=[pltpu.VMEM((2, 2 * _D), jnp.float32)],
    )(x, aggcat, aggcat, eps_p.reshape(1, 1), W1, b1.reshape(1, 2 * _D))


def _mlp2_body(h1_ref, sums_ref, g_ref, bt_ref, w2_ref, b2_ref, o_ref):
    mean = sums_ref[0:1, :] * (1.0 / _N)
    var = sums_ref[1:2, :] * (1.0 / _N) - mean * mean
    scale = g_ref[...] * lax.rsqrt(var + 1e-5)
    shift = bt_ref[...] - mean * scale
    h = jnp.maximum(h1_ref[...] * scale + shift, 0.0)
    o_ref[...] = (
        jnp.dot(h, w2_ref[...], preferred_element_type=jnp.float32) + b2_ref[...]
    )


def _mlp_stage2(h1, sums, gamma1, beta1, W2, b2):
    return pl.pallas_call(
        _mlp2_body,
        grid=(_N // _NB,),
        in_specs=[
            pl.BlockSpec((_NB, 2 * _D), lambda i: (i, 0)),
            pl.BlockSpec((2, 2 * _D), lambda i: (0, 0)),
            pl.BlockSpec((1, 2 * _D), lambda i: (0, 0)),
            pl.BlockSpec((1, 2 * _D), lambda i: (0, 0)),
            pl.BlockSpec((2 * _D, _D), lambda i: (0, 0)),
            pl.BlockSpec((1, _D), lambda i: (0, 0)),
        ],
        out_specs=pl.BlockSpec((_NB, _D), lambda i: (i, 0)),
        out_shape=jax.ShapeDtypeStruct((_N, _D), jnp.float32),
    )(
        h1,
        sums,
        gamma1.reshape(1, 2 * _D),
        beta1.reshape(1, 2 * _D),
        W2,
        b2.reshape(1, _D),
    )


def kernel(x, edge_index, edge_attr, W_e, b_e, eps_p, W1, b1, gamma1, beta1, W2, b2):
    src = edge_index[0]
    dst = edge_index[1]

    ecat = _edge_encoder(edge_attr, W_e, b_e).reshape(2 * _E, _H)
    xcat = jnp.concatenate([x[:, :_H], x[:, _H:]], axis=0)
    aggcat = _sc_aggregate(xcat, ecat, src, dst)
    h1, sums = _mlp_stage1(x, aggcat, eps_p, W1, b1)
    return _mlp_stage2(h1, sums, gamma1, beta1, W2, b2)


# speedup vs baseline: 2.0752x; 2.0752x over previous
import functools

import jax
import jax.numpy as jnp
from jax import lax
from jax.experimental import pallas as pl
from jax.experimental.pallas import tpu as pltpu
from jax.experimental.pallas import tpu_sc as plsc

_N = 10000
_E = 160000
_D = 256
_H = 128
_ED = 16

_EB = 8000
_NB = 1000

_NSUB = 16
_B = 80
_EPW = _E // _NSUB
_NCH = _EPW // _B
_RPT = 624
_TAIL = _N - _NSUB * _RPT
_ZR = 208


def _edge_encoder_body(ea_ref, we_ref, be_ref, out_ref):
    ee = jnp.dot(ea_ref[...], we_ref[...], preferred_element_type=jnp.float32)
    out_ref[0] = ee + be_ref[...]


def _edge_encoder(edge_attr, W_e, b_e):
    return pl.pallas_call(
        _edge_encoder_body,
        grid=(2, _E // _EB),
        in_specs=[
            pl.BlockSpec((_EB, _ED), lambda h, i: (i, 0)),
            pl.BlockSpec((_ED, _H), lambda h, i: (0, h)),
            pl.BlockSpec((1, _H), lambda h, i: (0, h)),
        ],
        out_specs=pl.BlockSpec((1, _EB, _H), lambda h, i: (h, i, 0)),
        out_shape=jax.ShapeDtypeStruct((2, _E, _H), jnp.float32),
    )(edge_attr, W_e, b_e.reshape(1, _D))


_sc_mesh = plsc.VectorSubcoreMesh(core_axis_name="c", subcore_axis_name="s")


@functools.partial(
    pl.kernel,
    mesh=_sc_mesh,
    out_type=jax.ShapeDtypeStruct((2 * _N, _H), jnp.float32),
    scratch_types=[
        pltpu.VMEM((_B,), jnp.int32),
        pltpu.VMEM((_B,), jnp.int32),
        pltpu.VMEM((_B, _H), jnp.float32),
        pltpu.VMEM((_B, _H), jnp.float32),
        pltpu.VMEM((_ZR, _H), jnp.float32),
        pltpu.VMEM_SHARED((_N, _H), jnp.float32),
        pltpu.SemaphoreType.DMA,
    ],
)
def _sc_aggregate(xcat, ecat, src, dst, out, srcv, dstv, gbuf, ebuf, zbuf, acc, sem):
    c = lax.axis_index("c")
    s = lax.axis_index("s")

    def _zrow(r, _):
        for j in range(_H // 16):
            zbuf[r, pl.ds(j * 16, 16)] = jnp.zeros((16,), jnp.float32)
        return 0

    lax.fori_loop(0, _ZR, _zrow, 0)
    for k in range(_RPT // _ZR):
        pltpu.sync_copy(zbuf, acc.at[pl.ds(s * _RPT + k * _ZR, _ZR)])

    @pl.when(s == _NSUB - 1)
    def _():
        pltpu.sync_copy(
            zbuf.at[pl.ds(0, _TAIL)], acc.at[pl.ds(_NSUB * _RPT, _TAIL)]
        )

    plsc.subcore_barrier()

    base_e = s * _EPW
    row_off = c * _N

    def _chunk(i, _):
        eb = base_e + i * _B
        pltpu.sync_copy(src.at[pl.ds(eb, _B)], srcv)
        pltpu.sync_copy(dst.at[pl.ds(eb, _B)], dstv)
        for j in range(_B // 16):
            sl = pl.ds(j * 16, 16)
            srcv[sl] = srcv[sl] + row_off
        gath = pltpu.async_copy(xcat.at[srcv], gbuf, sem)
        pltpu.sync_copy(ecat.at[pl.ds(c * _E + eb, _B)], ebuf)
        gath.wait()

        def _row(r, _):
            for j in range(_H // 16):
                sl = pl.ds(j * 16, 16)
                gbuf[r, sl] = jnp.maximum(gbuf[r, sl] + ebuf[r, sl], 0.0)
            return 0

        lax.fori_loop(0, _B, _row, 0)
        pltpu.sync_copy(gbuf, acc.at[dstv], add=True)
        return 0

    lax.fori_loop(0, _NCH, _chunk, 0)
    plsc.subcore_barrier()
    pltpu.sync_copy(
        acc.at[pl.ds(s * _RPT, _RPT)], out.at[pl.ds(row_off + s * _RPT, _RPT)]
    )

    @pl.when(s == _NSUB - 1)
    def _():
        pltpu.sync_copy(
            acc.at[pl.ds(_NSUB * _RPT, _TAIL)],
            out.at[pl.ds(row_off + _NSUB * _RPT, _TAIL)],
        )


def _mlp1_body(x_ref, a0_ref, a1_ref, eps_ref, w1_ref, b1_ref, h1_ref, sums_ref, acc_ref):
    i = pl.program_id(0)
    eps = eps_ref[0, 0]
    agg = jnp.concatenate([a0_ref[...], a1_ref[...]], axis=1)
    h = (1.0 + eps) * x_ref[...] + agg
    h1 = jnp.dot(h, w1_ref[...], preferred_element_type=jnp.float32) + b1_ref[...]
    h1_ref[...] = h1

    @pl.when(i == 0)
    def _():
        acc_ref[...] = jnp.zeros_like(acc_ref)

    acc_ref[0:1, :] += jnp.sum(h1, axis=0, keepdims=True)
    acc_ref[1:2, :] += jnp.sum(h1 * h1, axis=0, keepdims=True)

    @pl.when(i == pl.num_programs(0) - 1)
    def _():
        sums_ref[...] = acc_ref[...]


def _mlp_stage1(x, aggcat, eps_p, W1, b1):
    nb = _N // _NB
    return pl.pallas_call(
        _mlp1_body,
        grid=(nb,),
        in_specs=[
            pl.BlockSpec((_NB, _D), lambda i: (i, 0)),
            pl.BlockSpec((_NB, _H), lambda i: (i, 0)),
            pl.BlockSpec((_NB, _H), lambda i: (i + nb, 0)),
            pl.BlockSpec(memory_space=pltpu.SMEM),
            pl.BlockSpec((_D, 2 * _D), lambda i: (0, 0)),
            pl.BlockSpec((1, 2 * _D), lambda i: (0, 0)),
        ],
        out_specs=[
            pl.BlockSpec((_NB, 2 * _D), lambda i: (i, 0)),
            pl.BlockSpec((2, 2 * _D), lambda i: (0, 0)),
        ],
        out_shape=[
            jax.ShapeDtypeStruct((_N, 2 * _D), jnp.float32),
            jax.ShapeDtypeStruct((2, 2 * _D), jnp.float32),
        ],
        scratch_shapes=[pltpu.VMEM((2, 2 * _D), jnp.float32)],
    )(x, aggcat, aggcat, eps_p.reshape(1, 1), W1, b1.reshape(1, 2 * _D))


def _mlp2_body(h1_ref, sums_ref, g_ref, bt_ref, w2_ref, b2_ref, o_ref):
    mean = sums_ref[0:1, :] * (1.0 / _N)
    var = sums_ref[1:2, :] * (1.0 / _N) - mean * mean
    scale = g_ref[...] * lax.rsqrt(var + 1e-5)
    shift = bt_ref[...] - mean * scale
    h = jnp.maximum(h1_ref[...] * scale + shift, 0.0)
    o_ref[...] = (
        jnp.dot(h, w2_ref[...], preferred_element_type=jnp.float32) + b2_ref[...]
    )


def _mlp_stage2(h1, sums, gamma1, beta1, W2, b2):
    return pl.pallas_call(
        _mlp2_body,
        grid=(_N // _NB,),
        in_specs=[
            pl.BlockSpec((_NB, 2 * _D), lambda i: (i, 0)),
            pl.BlockSpec((2, 2 * _D), lambda i: (0, 0)),
            pl.BlockSpec((1, 2 * _D), lambda i: (0, 0)),
            pl.BlockSpec((1, 2 * _D), lambda i: (0, 0)),
            pl.BlockSpec((2 * _D, _D), lambda i: (0, 0)),
            pl.BlockSpec((1, _D), lambda i: (0, 0)),
        ],
        out_specs=pl.BlockSpec((_NB, _D), lambda i: (i, 0)),
        out_shape=jax.ShapeDtypeStruct((_N, _D), jnp.float32),
    )(
        h1,
        sums,
        gamma1.reshape(1, 2 * _D),
        beta1.reshape(1, 2 * _D),
        W2,
        b2.reshape(1, _D),
    )


def kernel(x, edge_index, edge_attr, W_e, b_e, eps_p, W1, b1, gamma1, beta1, W2, b2):
    src = edge_index[0]
    dst = edge_index[1]

    ecat = _edge_encoder(edge_attr, W_e, b_e).reshape(2 * _E, _H)
    xcat = jnp.concatenate([x[:, :_H], x[:, _H:]], axis=0)
    aggcat = _sc_aggregate(xcat, ecat, src, dst)
    h1, sums = _mlp_stage1(x, aggcat, eps_p, W1, b1)
    return _mlp_stage2(h1, sums, gamma1, beta1, W2, b2)


# speedup vs baseline: 3.3774x; 1.6275x over previous
import functools

import jax
import jax.numpy as jnp
from jax import lax
from jax.experimental import pallas as pl
from jax.experimental.pallas import tpu as pltpu
from jax.experimental.pallas import tpu_sc as plsc

_N = 10000
_E = 160000
_D = 256
_H = 128
_ED = 16

_EB = 8000
_NB = 1000

_NSUB = 16
_B = 80
_EPW = _E // _NSUB
_NCH = _EPW // _B
_RPT = 624
_TAIL = _N - _NSUB * _RPT
_ZR = 208


def _edge_encoder_body(ea_ref, we_ref, be_ref, out_ref):
    ee = jnp.dot(ea_ref[...], we_ref[...], preferred_element_type=jnp.float32)
    out_ref[0] = ee + be_ref[...]


def _edge_encoder(edge_attr, W_e, b_e):
    return pl.pallas_call(
        _edge_encoder_body,
        grid=(2, _E // _EB),
        in_specs=[
            pl.BlockSpec((_EB, _ED), lambda h, i: (i, 0)),
            pl.BlockSpec((_ED, _H), lambda h, i: (0, h)),
            pl.BlockSpec((1, _H), lambda h, i: (0, h)),
        ],
        out_specs=pl.BlockSpec((1, _EB, _H), lambda h, i: (h, i, 0)),
        out_shape=jax.ShapeDtypeStruct((2, _E, _H), jnp.float32),
    )(edge_attr, W_e, b_e.reshape(1, _D))


_sc_mesh = plsc.VectorSubcoreMesh(core_axis_name="c", subcore_axis_name="s")


@functools.partial(
    pl.kernel,
    mesh=_sc_mesh,
    out_type=jax.ShapeDtypeStruct((2 * _N, _H), jnp.float32),
    scratch_types=[
        pltpu.VMEM((_B,), jnp.int32),
        pltpu.VMEM((_B,), jnp.int32),
        pltpu.VMEM((_B,), jnp.int32),
        pltpu.VMEM((_B,), jnp.int32),
        pltpu.VMEM((_B,), jnp.int32),
        pltpu.VMEM((_B,), jnp.int32),
        pltpu.VMEM((_B,), jnp.int32),
        pltpu.VMEM((_B,), jnp.int32),
        pltpu.VMEM((_B, _H), jnp.float32),
        pltpu.VMEM((_B, _H), jnp.float32),
        pltpu.VMEM((_B, _H), jnp.float32),
        pltpu.VMEM((_B, _H), jnp.float32),
        pltpu.VMEM_SHARED((_N, _H), jnp.float32),
        pltpu.SemaphoreType.DMA,
        pltpu.SemaphoreType.DMA,
        pltpu.SemaphoreType.DMA,
        pltpu.SemaphoreType.DMA,
        pltpu.SemaphoreType.DMA,
        pltpu.SemaphoreType.DMA,
        pltpu.SemaphoreType.DMA,
        pltpu.SemaphoreType.DMA,
        pltpu.SemaphoreType.DMA,
        pltpu.SemaphoreType.DMA,
        pltpu.SemaphoreType.DMA,
        pltpu.SemaphoreType.DMA,
    ],
)
def _sc_aggregate(
    xcat, ecat, src, dst, out,
    srcv0, srcv1, srcv2, srcv3, dstv0, dstv1, dstv2, dstv3,
    gbuf0, gbuf1, ebuf0, ebuf1, acc,
    gsem0, gsem1, esem0, esem1,
    ssem0, ssem1, ssem2, ssem3, dsem0, dsem1, dsem2, dsem3,
):
    c = lax.axis_index("c")
    s = lax.axis_index("s")
    srcvs = (srcv0, srcv1, srcv2, srcv3)
    dstvs = (dstv0, dstv1, dstv2, dstv3)
    gbufs = (gbuf0, gbuf1)
    ebufs = (ebuf0, ebuf1)
    gsems = (gsem0, gsem1)
    esems = (esem0, esem1)
    ssems = (ssem0, ssem1, ssem2, ssem3)
    dsems = (dsem0, dsem1, dsem2, dsem3)

    base_e = s * _EPW
    row_off = c * _N

    def _idx_start(i, p):
        off = base_e + i * _B
        pltpu.async_copy(src.at[pl.ds(off, _B)], srcvs[p], ssems[p])
        pltpu.async_copy(dst.at[pl.ds(off, _B)], dstvs[p], dsems[p])

    def _idx_finish(i, p):
        off = base_e + i * _B
        pltpu.make_async_copy(src.at[pl.ds(off, _B)], srcvs[p], ssems[p]).wait()
        pltpu.make_async_copy(dst.at[pl.ds(off, _B)], dstvs[p], dsems[p]).wait()
        for j in range(_B // 16):
            sl = pl.ds(j * 16, 16)
            srcvs[p][sl] = srcvs[p][sl] + row_off

    def _gather_start(i, p4, p2):
        pltpu.async_copy(xcat.at[srcvs[p4]], gbufs[p2], gsems[p2])
        pltpu.async_copy(
            ecat.at[pl.ds(c * _E + base_e + i * _B, _B)], ebufs[p2], esems[p2]
        )

    def _compute_scatter(i, p4, p2):
        pltpu.make_async_copy(xcat.at[srcvs[p4]], gbufs[p2], gsems[p2]).wait()
        pltpu.make_async_copy(
            ecat.at[pl.ds(c * _E + base_e + i * _B, _B)], ebufs[p2], esems[p2]
        ).wait()
        gb, eb = gbufs[p2], ebufs[p2]

        def _row(r, _):
            for rr in range(2):
                for j in range(_H // 16):
                    sl = pl.ds(j * 16, 16)
                    gb[2 * r + rr, sl] = jnp.maximum(
                        gb[2 * r + rr, sl] + eb[2 * r + rr, sl], 0.0
                    )
            return 0

        lax.fori_loop(0, _B // 2, _row, 0)
        pltpu.sync_copy(gb, acc.at[dstvs[p4]], add=True)

    _idx_start(0, 0)
    _idx_start(1, 1)
    _idx_start(2, 2)

    def _zrow(r, _):
        for j in range(_H // 16):
            ebuf0[r, pl.ds(j * 16, 16)] = jnp.zeros((16,), jnp.float32)
        return 0

    lax.fori_loop(0, _B, _zrow, 0)
    for k in range(_RPT // _B):
        pltpu.sync_copy(ebuf0, acc.at[pl.ds(s * _RPT + k * _B, _B)])
    _zrem = _RPT - (_RPT // _B) * _B
    if _zrem:
        pltpu.sync_copy(
            ebuf0.at[pl.ds(0, _zrem)],
            acc.at[pl.ds(s * _RPT + (_RPT // _B) * _B, _zrem)],
        )

    @pl.when(s == _NSUB - 1)
    def _():
        pltpu.sync_copy(
            ebuf0.at[pl.ds(0, _TAIL)], acc.at[pl.ds(_NSUB * _RPT, _TAIL)]
        )

    plsc.subcore_barrier()

    _idx_finish(0, 0)
    _gather_start(0, 0, 0)

    def _quad(k, _):
        i0 = 4 * k
        for t in range(4):
            i = i0 + t
            _idx_finish(i + 1, (t + 1) % 4)
            _gather_start(i + 1, (t + 1) % 4, (t + 1) % 2)
            _compute_scatter(i, t % 4, t % 2)

            @pl.when(i + 3 < _NCH)
            def _():
                _idx_start(i + 3, (t + 3) % 4)

        return 0

    lax.fori_loop(0, (_NCH - 1) // 4, _quad, 0)
    _compute_scatter(_NCH - 1, (_NCH - 1) % 4, (_NCH - 1) % 2)
    plsc.subcore_barrier()
    pltpu.sync_copy(
        acc.at[pl.ds(s * _RPT, _RPT)], out.at[pl.ds(row_off + s * _RPT, _RPT)]
    )

    @pl.when(s == _NSUB - 1)
    def _():
        pltpu.sync_copy(
            acc.at[pl.ds(_NSUB * _RPT, _TAIL)],
            out.at[pl.ds(row_off + _NSUB * _RPT, _TAIL)],
        )


def _mlp_body(
    x_ref, a0_ref, a1_ref, eps_ref, w1_ref, b1_ref, g_ref, bt_ref, w2_ref,
    b2_ref, o_ref, h1_ref, acc_ref,
):
    p = pl.program_id(0)
    i = pl.program_id(1)

    @pl.when(p == 0)
    def _():
        eps = eps_ref[0, 0]
        agg = jnp.concatenate([a0_ref[...], a1_ref[...]], axis=1)
        h = (1.0 + eps) * x_ref[...] + agg
        h1 = (
            jnp.dot(h, w1_ref[...], preferred_element_type=jnp.float32)
            + b1_ref[...]
        )
        h1_ref[pl.ds(i * _NB, _NB), :] = h1

        @pl.when(i == 0)
        def _():
            acc_ref[...] = jnp.zeros_like(acc_ref)

        acc_ref[0:1, :] += jnp.sum(h1, axis=0, keepdims=True)
        acc_ref[1:2, :] += jnp.sum(h1 * h1, axis=0, keepdims=True)

    @pl.when(p == 1)
    def _():
        mean = acc_ref[0:1, :] * (1.0 / _N)
        var = acc_ref[1:2, :] * (1.0 / _N) - mean * mean
        scale = g_ref[...] * lax.rsqrt(var + 1e-5)
        shift = bt_ref[...] - mean * scale
        h1 = h1_ref[pl.ds(i * _NB, _NB), :]
        hr = jnp.maximum(h1 * scale + shift, 0.0)
        o_ref[...] = (
            jnp.dot(hr, w2_ref[...], preferred_element_type=jnp.float32)
            + b2_ref[...]
        )


def _mlp(x, aggcat, eps_p, W1, b1, gamma1, beta1, W2, b2):
    nb = _N // _NB
    return pl.pallas_call(
        _mlp_body,
        grid=(2, nb),
        in_specs=[
            pl.BlockSpec((_NB, _D), lambda p, i: (i, 0)),
            pl.BlockSpec((_NB, _H), lambda p, i: (i, 0)),
            pl.BlockSpec((_NB, _H), lambda p, i: (i + nb, 0)),
            pl.BlockSpec(memory_space=pltpu.SMEM),
            pl.BlockSpec((_D, 2 * _D), lambda p, i: (0, 0)),
            pl.BlockSpec((1, 2 * _D), lambda p, i: (0, 0)),
            pl.BlockSpec((1, 2 * _D), lambda p, i: (0, 0)),
            pl.BlockSpec((1, 2 * _D), lambda p, i: (0, 0)),
            pl.BlockSpec((2 * _D, _D), lambda p, i: (0, 0)),
            pl.BlockSpec((1, _D), lambda p, i: (0, 0)),
        ],
        out_specs=pl.BlockSpec((_NB, _D), lambda p, i: (i, 0)),
        out_shape=jax.ShapeDtypeStruct((_N, _D), jnp.float32),
        scratch_shapes=[
            pltpu.VMEM((_N, 2 * _D), jnp.float32),
            pltpu.VMEM((2, 2 * _D), jnp.float32),
        ],
    )(
        x, aggcat, aggcat, eps_p.reshape(1, 1), W1, b1.reshape(1, 2 * _D),
        gamma1.reshape(1, 2 * _D), beta1.reshape(1, 2 * _D), W2,
        b2.reshape(1, _D),
    )


def kernel(x, edge_index, edge_attr, W_e, b_e, eps_p, W1, b1, gamma1, beta1, W2, b2):
    src = edge_index[0]
    dst = edge_index[1]

    ecat = _edge_encoder(edge_attr, W_e, b_e).reshape(2 * _E, _H)
    xcat = jnp.concatenate([x[:, :_H], x[:, _H:]], axis=0)
    aggcat = _sc_aggregate(xcat, ecat, src, dst)
    return _mlp(x, aggcat, eps_p, W1, b1, gamma1, beta1, W2, b2)
